# baseline (speedup 1.0000x reference)
import functools

import jax
import jax.numpy as jnp
from jax import lax
from jax.experimental import pallas as pl
from jax.experimental.pallas import tpu as pltpu
from jax.experimental.pallas import tpu_sc as plsc

B_ROWS = 16384
B_COLS = 200
D = 24
TPAD = 25
B = B_ROWS * B_COLS
NC, NS = 2, 16
NW = NC * NS
ROWS_PER_STEP = 1024
GROUPS = ROWS_PER_STEP // 16
B_PER_W = B // NW
STEPS = B_PER_W // ROWS_PER_STEP


def _sc_lookup(idx_flat, tflat):
    mesh = plsc.VectorSubcoreMesh(core_axis_name="c", subcore_axis_name="s")

    @functools.partial(
        pl.kernel,
        mesh=mesh,
        compiler_params=pltpu.CompilerParams(
            use_tc_tiling_on_sc=False, needs_layout_passes=False),
        out_type=jax.ShapeDtypeStruct((B, D), jnp.float32),
        scratch_types=[
            pltpu.VMEM((D * TPAD,), jnp.float32),
            pltpu.VMEM((ROWS_PER_STEP,), jnp.int32),
            pltpu.VMEM((ROWS_PER_STEP,), jnp.int32),
            pltpu.VMEM((ROWS_PER_STEP, TPAD), jnp.float32),
            pltpu.VMEM((ROWS_PER_STEP, TPAD), jnp.float32),
            pltpu.SemaphoreType.DMA,
            pltpu.SemaphoreType.DMA,
            pltpu.SemaphoreType.DMA,
            pltpu.SemaphoreType.DMA,
        ],
    )
    def k(idx_hbm, tab_hbm, out_hbm, tab_v,
          idx_v0, idx_v1, out_v0, out_v1, si0, si1, so0, so1):
        wid = lax.axis_index("s") * NC + lax.axis_index("c")
        row0 = wid * B_PER_W
        pltpu.sync_copy(tab_hbm, tab_v)
        iota = lax.iota(jnp.int32, 16)

        idx_bufs = (idx_v0, idx_v1)
        out_bufs = (out_v0, out_v1)
        si = (si0, si1)
        so = (so0, so1)

        def idx_slice(it):
            base = pl.multiple_of(row0 + it * ROWS_PER_STEP, 8)
            return idx_hbm.at[pl.ds(base, ROWS_PER_STEP)]

        def out_slice(it):
            base = pl.multiple_of(row0 + it * ROWS_PER_STEP, 8)
            return out_hbm.at[pl.ds(base, ROWS_PER_STEP)]

        def stage_view(ob):
            return ob.at[:, pl.ds(0, D)]

        pltpu.async_copy(idx_slice(0), idx_v0, si0)
        pltpu.async_copy(idx_slice(1), idx_v1, si1)

        def outer(i, carry):
            for b in range(2):
                it = 2 * i + b
                ib, ob, sib, sob = idx_bufs[b], out_bufs[b], si[b], so[b]
                pltpu.make_async_copy(idx_slice(it), ib, sib).wait()

                @pl.when(i > 0)
                def _wait_out():
                    pltpu.make_async_copy(
                        stage_view(ob), out_slice(it - 2), sob).wait()

                def group(g, c):
                    idxv = ib[pl.ds(g * 16, 16)]
                    gbase = idxv * TPAD
                    rowv = iota + g * 16
                    for kk in range(D):
                        val = plsc.load_gather(tab_v, [gbase + kk])
                        colv = jnp.full((16,), kk, jnp.int32)
                        plsc.store_scatter(ob, [rowv, colv], val)
                    return c

                lax.fori_loop(0, GROUPS, group, 0)
                pltpu.async_copy(stage_view(ob), out_slice(it), sob)

                @pl.when(it + 2 < STEPS)
                def _next_idx():
                    pltpu.async_copy(idx_slice(it + 2), ib, sib)
            return carry

        lax.fori_loop(0, STEPS // 2, outer, 0)
        pltpu.make_async_copy(stage_view(out_v0), out_slice(STEPS - 2), so0).wait()
        pltpu.make_async_copy(stage_view(out_v1), out_slice(STEPS - 1), so1).wait()

    return k(idx_flat, tflat)


def kernel(key_int_tensor, table):
    tpad = jnp.zeros((D, TPAD), jnp.float32).at[:, :D].set(table[:D, :])
    out = _sc_lookup(key_int_tensor.reshape(B), tpad.reshape(D * TPAD))
    return out.reshape(B_ROWS, B_COLS, D)


# speedup vs baseline: 1.6640x; 1.6640x over previous
import functools

import jax
import jax.numpy as jnp
from jax import lax
from jax.experimental import pallas as pl
from jax.experimental.pallas import tpu as pltpu
from jax.experimental.pallas import tpu_sc as plsc

B_ROWS = 16384
B_COLS = 200
D = 24
TPAD = 32
B = B_ROWS * B_COLS
NC, NS = 2, 16
NW = NC * NS
ROWS_PER_STEP = 2048
GROUPS = ROWS_PER_STEP // 16
OUT_PER_STEP = ROWS_PER_STEP * D
B_PER_W = B // NW
STEPS = B_PER_W // ROWS_PER_STEP


def _sc_lookup(idx_flat, tflat):
    mesh = plsc.VectorSubcoreMesh(core_axis_name="c", subcore_axis_name="s")

    @functools.partial(
        pl.kernel,
        mesh=mesh,
        compiler_params=pltpu.CompilerParams(
            use_tc_tiling_on_sc=False, needs_layout_passes=False),
        out_type=jax.ShapeDtypeStruct((B * D,), jnp.float32),
        scratch_types=[
            pltpu.VMEM((D * TPAD,), jnp.float32),
            pltpu.VMEM((ROWS_PER_STEP,), jnp.int32),
            pltpu.VMEM((ROWS_PER_STEP,), jnp.int32),
            pltpu.VMEM((OUT_PER_STEP + 16,), jnp.float32),
            pltpu.VMEM((OUT_PER_STEP + 16,), jnp.float32),
            pltpu.SemaphoreType.DMA,
            pltpu.SemaphoreType.DMA,
            pltpu.SemaphoreType.DMA,
            pltpu.SemaphoreType.DMA,
        ],
    )
    def k(idx_hbm, tab_hbm, out_hbm, tab_v,
          idx_v0, idx_v1, out_v0, out_v1, si0, si1, so0, so1):
        wid = lax.axis_index("s") * NC + lax.axis_index("c")
        row0 = wid * B_PER_W
        pltpu.sync_copy(tab_hbm, tab_v)

        idx_bufs = (idx_v0, idx_v1)
        out_bufs = (out_v0, out_v1)
        si = (si0, si1)
        so = (so0, so1)

        def idx_slice(it):
            base = pl.multiple_of(row0 + it * ROWS_PER_STEP, 8)
            return idx_hbm.at[pl.ds(base, ROWS_PER_STEP)]

        def out_slice(it):
            base = pl.multiple_of((row0 + it * ROWS_PER_STEP) * D, 8)
            return out_hbm.at[pl.ds(base, OUT_PER_STEP)]

        def stage_view(ob):
            return ob.at[pl.ds(0, OUT_PER_STEP)]

        pltpu.async_copy(idx_slice(0), idx_v0, si0)
        pltpu.async_copy(idx_slice(1), idx_v1, si1)

        def outer(i, carry):
            for b in range(2):
                it = 2 * i + b
                ib, ob, sib, sob = idx_bufs[b], out_bufs[b], si[b], so[b]
                pltpu.make_async_copy(idx_slice(it), ib, sib).wait()

                @pl.when(i > 0)
                def _wait_out():
                    pltpu.make_async_copy(
                        stage_view(ob), out_slice(it - 2), sob).wait()

                def group(g, c):
                    obase = g * (16 * D)
                    idxv = ib[pl.ds(g * 16, 16)]
                    for u in range(16):
                        a = idxv[u] * TPAD
                        v1 = tab_v[pl.ds(a, 16)]
                        v2 = tab_v[pl.ds(a + 8, 16)]
                        ob[pl.ds(obase + u * D, 16)] = v1
                        ob[pl.ds(obase + u * D + 8, 16)] = v2
                    return c

                lax.fori_loop(0, GROUPS, group, 0)
                pltpu.async_copy(stage_view(ob), out_slice(it), sob)

                @pl.when(it + 2 < STEPS)
                def _next_idx():
                    pltpu.async_copy(idx_slice(it + 2), ib, sib)
            return carry

        lax.fori_loop(0, STEPS // 2, outer, 0)
        pltpu.make_async_copy(stage_view(out_v0), out_slice(STEPS - 2), so0).wait()
        pltpu.make_async_copy(stage_view(out_v1), out_slice(STEPS - 1), so1).wait()

    return k(idx_flat, tflat)


def kernel(key_int_tensor, table):
    tpad = jnp.zeros((D, TPAD), jnp.float32).at[:, :D].set(table[:D, :])
    out = _sc_lookup(key_int_tensor.reshape(B), tpad.reshape(D * TPAD))
    return out.reshape(B_ROWS, B_COLS, D)
